# baseline (speedup 1.0000x reference)
import functools
import jax
import jax.numpy as jnp
from jax import lax
from jax.experimental import pallas as pl
from jax.experimental.pallas import tpu as pltpu, tpu_sc as plsc

_E = 16
_C = 768
_WPB = 442368
_CHUNK = 18432
_NCHUNK = 24
_NB_PER = 2

_GDN = lax.GatherDimensionNumbers(
    offset_dims=(), collapsed_slice_dims=(0,), start_index_map=(0,))


def _perm(vec, idx):
    return lax.gather(vec, idx[:, None], _GDN, (1,),
                      mode=lax.GatherScatterMode.PROMISE_IN_BOUNDS)


def _pool_sc(x1, B):
    mesh = plsc.VectorSubcoreMesh(core_axis_name="c", subcore_axis_name="s")

    @functools.partial(
        pl.kernel, mesh=mesh,
        out_type=jax.ShapeDtypeStruct((B, _C), jnp.float32),
        scratch_types=[
            pltpu.VMEM((_CHUNK,), jnp.float32),
            pltpu.VMEM((_CHUNK,), jnp.float32),
            pltpu.VMEM((_C,), jnp.float32),
            pltpu.SemaphoreType.DMA,
            pltpu.SemaphoreType.DMA,
        ],
    )
    def k(x_hbm, out_hbm, buf0, buf1, csum, sem0, sem1):
        cid = lax.axis_index("c")
        sid = lax.axis_index("s")
        wid = sid * 2 + cid

        lane = lax.iota(jnp.int32, 16)
        pair_idx = lane ^ 1
        quad_idx = lane ^ 2
        quad_mask = (lane % 4) == 0

        bufs = (buf0, buf1)
        sems = (sem0, sem1)

        for bi in range(_NB_PER):
            bidx = wid * _NB_PER + bi
            base = bidx * _WPB

            pltpu.make_async_copy(
                x_hbm.at[pl.ds(base, _CHUNK)], buf0, sem0).start()
            for ch in range(_NCHUNK):
                cur = bufs[ch % 2]
                if ch + 1 < _NCHUNK:
                    pltpu.make_async_copy(
                        x_hbm.at[pl.ds(base + (ch + 1) * _CHUNK, _CHUNK)],
                        bufs[(ch + 1) % 2], sems[(ch + 1) % 2]).start()
                pltpu.make_async_copy(
                    x_hbm.at[pl.ds(base + ch * _CHUNK, _CHUNK)],
                    cur, sems[ch % 2]).wait()

                def kbody(k, accs):
                    return tuple(
                        accs[j] + cur[pl.ds(576 * j + 16 * k, 16)]
                        for j in range(32))

                zero = jnp.zeros((16,), jnp.float32)
                accs = lax.fori_loop(0, 36, kbody, (zero,) * 32)

                def combine(a, bvec, sbit):
                    return jnp.where((lane & sbit) == 0,
                                     a + _perm(a, lane ^ sbit),
                                     bvec + _perm(bvec, lane ^ sbit))

                for half in range(2):
                    vs = list(accs[half * 16:(half + 1) * 16])
                    sbit = 1
                    while len(vs) > 1:
                        vs = [combine(vs[2 * i], vs[2 * i + 1], sbit)
                              for i in range(len(vs) // 2)]
                        sbit *= 2
                    csum[pl.ds(ch * 32 + half * 16, 16)] = vs[0]

            pltpu.sync_copy(csum, out_hbm.at[bidx])

    return k(x1)


def _gate_tc(csum, wt, b2):
    B = csum.shape[0]

    def body(s_ref, wt_ref, b_ref, out_ref):
        logits = jnp.dot(s_ref[...], wt_ref[...],
                         precision=jax.lax.Precision.HIGHEST,
                         preferred_element_type=jnp.float32)
        logits = logits * (1.0 / 576.0) + b_ref[...]
        iota = jax.lax.broadcasted_iota(jnp.int32, (B, _E), 1)
        m = jnp.max(logits, axis=1, keepdims=True)
        first = jnp.min(jnp.where(logits == m, iota, _E),
                        axis=1, keepdims=True)
        out_ref[...] = (iota == first).astype(jnp.float32)

    return pl.pallas_call(
        body,
        out_shape=jax.ShapeDtypeStruct((B, _E), jnp.float32),
    )(csum, wt, b2)


def kernel(x, W, b):
    B = x.shape[0]
    x1 = x.reshape(B * _WPB)
    csum = _pool_sc(x1, B)
    return _gate_tc(csum, W.T, b.reshape(1, _E))


# speedup vs baseline: 4.7740x; 4.7740x over previous
import jax
import jax.numpy as jnp
from jax.experimental import pallas as pl

_E = 16
_C = 768
_HW = 576
_BB = 2


def _gate_kernel(x_ref, wt_ref, b_ref, out_ref):
    for i in range(_BB):
        xb = x_ref[i]
        s = jnp.sum(xb, axis=1, keepdims=True)
        prod = s * wt_ref[...]
        logits = (jnp.sum(prod, axis=0, keepdims=True) * (1.0 / _HW)
                  + b_ref[...])
        iota = jax.lax.broadcasted_iota(jnp.int32, (1, _E), 1)
        m = jnp.max(logits, axis=1, keepdims=True)
        first = jnp.min(jnp.where(logits == m, iota, _E),
                        axis=1, keepdims=True)
        out_ref[i] = (iota == first).astype(jnp.float32)


def kernel(x, W, b):
    B = x.shape[0]
    x3 = x.reshape(B, _C, _HW)
    wt = W.T
    b2 = b.reshape(1, _E)
    out = pl.pallas_call(
        _gate_kernel,
        grid=(B // _BB,),
        in_specs=[
            pl.BlockSpec((_BB, _C, _HW), lambda i: (i, 0, 0)),
            pl.BlockSpec((_C, _E), lambda i: (0, 0)),
            pl.BlockSpec((1, _E), lambda i: (0, 0)),
        ],
        out_specs=pl.BlockSpec((_BB, 1, _E), lambda i: (i, 0, 0)),
        out_shape=jax.ShapeDtypeStruct((B, 1, _E), jnp.float32),
    )(x3, wt, b2)
    return out.reshape(B, _E)


# speedup vs baseline: 5.0357x; 1.0548x over previous
import jax
import jax.numpy as jnp
from jax.experimental import pallas as pl

_E = 16
_C = 768
_HW = 576
_BB = 4


def _gate_kernel(x_ref, wt_ref, b_ref, out_ref):
    for i in range(_BB):
        xb = x_ref[i]
        s = jnp.sum(xb, axis=1, keepdims=True)
        prod = s * wt_ref[...]
        logits = (jnp.sum(prod, axis=0, keepdims=True) * (1.0 / _HW)
                  + b_ref[...])
        iota = jax.lax.broadcasted_iota(jnp.int32, (1, _E), 1)
        m = jnp.max(logits, axis=1, keepdims=True)
        first = jnp.min(jnp.where(logits == m, iota, _E),
                        axis=1, keepdims=True)
        out_ref[i] = (iota == first).astype(jnp.float32)


def kernel(x, W, b):
    B = x.shape[0]
    x3 = x.reshape(B, _C, _HW)
    wt = W.T
    b2 = b.reshape(1, _E)
    out = pl.pallas_call(
        _gate_kernel,
        grid=(B // _BB,),
        in_specs=[
            pl.BlockSpec((_BB, _C, _HW), lambda i: (i, 0, 0)),
            pl.BlockSpec((_C, _E), lambda i: (0, 0)),
            pl.BlockSpec((1, _E), lambda i: (0, 0)),
        ],
        out_specs=pl.BlockSpec((_BB, 1, _E), lambda i: (i, 0, 0)),
        out_shape=jax.ShapeDtypeStruct((B, 1, _E), jnp.float32),
    )(x3, wt, b2)
    return out.reshape(B, _E)
